# Initial kernel scaffold; baseline (speedup 1.0000x reference)
import jax
import jax.numpy as jnp
from jax.experimental import pallas as pl


def kernel(x, W_pre, b_pre, W_wp, b_wp, codebook, W_post, b_post):
    raise NotImplementedError("write your pallas kernel here")


# speedup vs baseline: 9.6218x; 9.6218x over previous
import jax
import jax.numpy as jnp
from jax.experimental import pallas as pl

_GROUPS = 8
_NUM_VARS = 64
_VAR_DIM = 64
_PROJ_DIM = 32


def _fused_body(x_ref, wpre_ref, bpre_ref, wwp_ref, bwp_ref, cb_ref,
                wpost_ref, bpost_ref, out_ref):
    x = x_ref[...]
    h = jnp.dot(x, wpre_ref[...], preferred_element_type=jnp.float32)
    h = h + bpre_ref[...]
    logits = jnp.dot(h, wwp_ref[...], preferred_element_type=jnp.float32)
    logits = logits + bwp_ref[...]
    rows = x.shape[0]
    iota = jax.lax.broadcasted_iota(jnp.int32, (rows, _NUM_VARS), 1)
    pieces = []
    for g in range(_GROUPS):
        sub = logits[:, g * _NUM_VARS:(g + 1) * _NUM_VARS]
        k = jnp.argmax(sub, axis=-1)
        oh = (iota == k[:, None]).astype(jnp.float32)
        pieces.append(jnp.dot(oh, cb_ref[g * _NUM_VARS:(g + 1) * _NUM_VARS, :],
                              preferred_element_type=jnp.float32))
    q = jnp.concatenate(pieces, axis=-1)
    out = jnp.dot(q, wpost_ref[...], preferred_element_type=jnp.float32)
    out_ref[...] = out + bpost_ref[...]


def kernel(x, W_pre, b_pre, W_wp, b_wp, codebook, W_post, b_post):
    B, T, IN_DIM = x.shape
    OUT_DIM = W_post.shape[1]
    BT = B * T
    BLK = 512
    xf = x.reshape(BT, IN_DIM)
    out = pl.pallas_call(
        _fused_body,
        grid=(BT // BLK,),
        in_specs=[
            pl.BlockSpec((BLK, IN_DIM), lambda i: (i, 0)),
            pl.BlockSpec((IN_DIM, _PROJ_DIM), lambda i: (0, 0)),
            pl.BlockSpec((1, _PROJ_DIM), lambda i: (0, 0)),
            pl.BlockSpec((_PROJ_DIM, _GROUPS * _NUM_VARS), lambda i: (0, 0)),
            pl.BlockSpec((1, _GROUPS * _NUM_VARS), lambda i: (0, 0)),
            pl.BlockSpec((_GROUPS * _NUM_VARS, _VAR_DIM), lambda i: (0, 0)),
            pl.BlockSpec((_GROUPS * _VAR_DIM, OUT_DIM), lambda i: (0, 0)),
            pl.BlockSpec((1, OUT_DIM), lambda i: (0, 0)),
        ],
        out_specs=pl.BlockSpec((BLK, OUT_DIM), lambda i: (i, 0)),
        out_shape=jax.ShapeDtypeStruct((BT, OUT_DIM), jnp.float32),
    )(xf, W_pre, b_pre.reshape(1, -1), W_wp, b_wp.reshape(1, -1),
      codebook, W_post, b_post.reshape(1, -1))
    return out.reshape(B, T, OUT_DIM)
